# Initial kernel scaffold; baseline (speedup 1.0000x reference)
import jax
import jax.numpy as jnp
from jax.experimental import pallas as pl


def kernel(x, edge_index, W1, b1, W2, b2, Wlin, blin):
    raise NotImplementedError("write your pallas kernel here")


# speedup vs baseline: 5.3289x; 5.3289x over previous
import functools

import jax
import jax.numpy as jnp
from jax import lax
from jax.experimental import pallas as pl
from jax.experimental.pallas import tpu as pltpu
from jax.experimental.pallas import tpu_sc as plsc

N = 10000
E = 160000
H = 640
HF = 384
NP = 10240
NC, NS = 2, 16
NW = NC * NS
R = NP // 64
CH = 1600
EPW = E // NW
HROWS = EPW // 128

_mesh = plsc.VectorSubcoreMesh(core_axis_name="c", subcore_axis_name="s")


@functools.partial(
    pl.kernel,
    mesh=_mesh,
    compiler_params=pltpu.CompilerParams(needs_layout_passes=False),
    out_type=jax.ShapeDtypeStruct((NC, NP), jnp.float32),
    scratch_types=[
        pltpu.VMEM((HROWS + 1, 128), jnp.int32),
        pltpu.VMEM((128,), jnp.float32),
        pltpu.VMEM((1024,), jnp.float32),
        pltpu.VMEM_SHARED((NP,), jnp.float32),
        pltpu.SemaphoreType.DMA,
    ],
)
def _deg_kernel(col_hbm, out_hbm, idx, ones_v, zeros_v, hist, sem):
    cid = lax.axis_index("c")
    sid = lax.axis_index("s")
    wid = sid * NC + cid
    base = wid * EPW

    one = jnp.ones((16,), jnp.float32)
    zero = jnp.zeros((16,), jnp.float32)
    for q in range(8):
        ones_v[pl.ds(q * 16, 16)] = one
    for q in range(64):
        zeros_v[pl.ds(q * 16, 16)] = zero
    pad = jnp.full((16,), N, jnp.int32)
    for q in range(8):
        idx[HROWS, pl.ds(q * 16, 16)] = pad

    @pl.when(sid == 0)
    def _():
        for k in range(NP // 1024):
            pltpu.sync_copy(zeros_v, hist.at[pl.ds(k * 1024, 1024)])

    for j in range(HROWS):
        pltpu.async_copy(col_hbm.at[pl.ds(base + j * 128, 128)], idx.at[j], sem)
    for j in range(HROWS):
        pltpu.make_async_copy(col_hbm.at[pl.ds(0, 128)], idx.at[0], sem).wait()
    pltpu.sync_copy(col_hbm.at[pl.ds(base + HROWS * 128, 8)],
                    idx.at[HROWS, pl.ds(0, 8)])

    plsc.subcore_barrier()
    for j in range(HROWS + 1):
        pltpu.sync_copy(ones_v, hist.at[idx.at[j]], add=True)
    plsc.subcore_barrier()

    @pl.when(sid == 0)
    def _():
        pltpu.sync_copy(hist, out_hbm.at[cid])


G = 16
NRING = 4
CAPE = 4096
FLUSH_AT = CAPE - CH - 24
NCHUNK = E // CH


@functools.partial(
    pl.kernel,
    mesh=_mesh,
    compiler_params=pltpu.CompilerParams(needs_layout_passes=False),
    out_type=jax.ShapeDtypeStruct((NP, HF), jnp.float32),
    scratch_types=[
        pltpu.VMEM((R, HF), jnp.float32),
        pltpu.VMEM((CH,), jnp.int32),
        pltpu.VMEM((CH,), jnp.int32),
        pltpu.VMEM((CH,), jnp.int32),
        pltpu.VMEM((CH,), jnp.int32),
        pltpu.VMEM((CAPE + 32,), jnp.int32),
        pltpu.VMEM((CAPE + 32,), jnp.int32),
        pltpu.VMEM((NRING * G, HF), jnp.float32),
        pltpu.SemaphoreType.DMA,
        pltpu.SemaphoreType.DMA,
        pltpu.SemaphoreType.DMA,
        pltpu.SemaphoreType.DMA,
        pltpu.SemaphoreType.DMA,
        pltpu.SemaphoreType.DMA,
    ],
)
def _segmax_kernel(u_hbm, col_hbm, row_hbm, m_hbm,
                   acc, colb0, colb1, rowb0, rowb1, dstb, srcb, msg,
                   semc0, semc1, semm0, semm1, semm2, semm3):
    cid = lax.axis_index("c")
    sid = lax.axis_index("s")
    wid = sid * NC + cid
    lanes = lax.broadcasted_iota(jnp.int32, (16,), 0)
    colb = (colb0, colb1)
    rowb = (rowb0, rowb1)
    semc = (semc0, semc1)
    semm = (semm0, semm1, semm2, semm3)

    def issue_chunk(idx, b):
        pltpu.async_copy(col_hbm.at[pl.ds(idx * CH, CH)], colb[b], semc[b])
        pltpu.async_copy(row_hbm.at[pl.ds(idx * CH, CH)], rowb[b], semc[b])

    def wait_chunk(b):
        pltpu.make_async_copy(col_hbm.at[pl.ds(0, CH)], colb[b], semc[b]).wait()
        pltpu.make_async_copy(row_hbm.at[pl.ds(0, CH)], rowb[b], semc[b]).wait()

    def issue_group(g):
        for par in range(NRING):
            @pl.when(lax.rem(g, NRING) == par)
            def _():
                pltpu.async_copy(u_hbm.at[srcb.at[pl.ds(g * G, G)]],
                                 msg.at[pl.ds(par * G, G)], semm[par])

    def drain_group(g):
        for par in range(NRING):
            @pl.when(lax.rem(g, NRING) == par)
            def _():
                pltpu.make_async_copy(u_hbm.at[pl.ds(0, G)],
                                      msg.at[pl.ds(par * G, G)],
                                      semm[par]).wait()

    def update_group(g, nj):
        dvec = dstb[pl.ds(g * G, 16)]
        mbase = lax.rem(g, NRING) * G

        def upd(j, _):
            d = jnp.max(jnp.where(lanes == j, dvec, -1))
            mj = mbase + j
            for r in range(HF // 16):
                sl = pl.ds(r * 16, 16)
                a = plsc.bitcast(acc[d, sl], jnp.bfloat16)
                m = plsc.bitcast(msg[mj, sl], jnp.bfloat16)
                acc[d, sl] = plsc.bitcast(jnp.maximum(a, m), jnp.float32)
            return 0

        lax.fori_loop(0, nj, upd, 0)

    def process_full(ng):
        for k in range(NRING - 1):
            @pl.when(ng > k)
            def _():
                issue_group(k)

        def gb(g, _):
            @pl.when(g + (NRING - 1) < ng)
            def _():
                issue_group(g + (NRING - 1))
            drain_group(g)
            update_group(g, 8)
            return 0

        lax.fori_loop(0, ng, gb, 0)

    for half in range(2):
        item = wid + NW * half
        base = item * R
        pltpu.sync_copy(u_hbm.at[pl.ds(base, R)], acc)
        issue_chunk(0, 0)

        def pair_body(c, cntv):
            for b in range(2):
                idx = 2 * c + b
                wait_chunk(b)

                @pl.when(idx + 1 < NCHUNK)
                def _():
                    issue_chunk(idx + 1, 1 - b)

                cb, rb = colb[b], rowb[b]

                def scan_body(v, cntv):
                    cv = cb[pl.ds(v * 16, 16)]
                    rv = rb[pl.ds(v * 16, 16)]
                    hit = (cv >= base) & (cv < base + R)
                    pf = plsc.cumsum(hit.astype(jnp.int32))
                    pos = cntv + pf - 1
                    plsc.store_scatter(dstb, [pos], cv - base, mask=hit)
                    plsc.store_scatter(srcb, [pos], rv, mask=hit)
                    return cntv + plsc.all_reduce_population_count(hit)

                cntv = lax.fori_loop(0, CH // 16, scan_body, cntv, unroll=4)
                cmax = jnp.max(cntv)

                def flush(cntv, cmax):
                    ng = cmax // G
                    process_full(ng)
                    sv = srcb[pl.ds(ng * G, 16)]
                    dv = dstb[pl.ds(ng * G, 16)]
                    srcb[pl.ds(0, 16)] = sv
                    dstb[pl.ds(0, 16)] = dv
                    return cntv - ng * G

                cntv = lax.cond(cmax > FLUSH_AT, flush,
                                lambda cntv, cmax: cntv, cntv, cmax)
            return cntv

        cntv = lax.fori_loop(0, NCHUNK // 2, pair_body,
                             jnp.zeros((16,), jnp.int32))

        cnt = jnp.max(cntv)
        ng = cnt // G
        rem = cnt - ng * G
        process_full(ng)
        srcb[pl.ds(cnt, 16)] = jnp.zeros((16,), jnp.int32)

        @pl.when(rem > 0)
        def _():
            issue_group(ng)
            drain_group(ng)
            update_group(ng, rem)

        pltpu.sync_copy(acc, m_hbm.at[pl.ds(base, R)])


_BLK = 1024
_GRID = NP // _BLK


def _pad_bf16(x):
    xb = x.astype(jnp.bfloat16)
    return jnp.concatenate(
        [xb, jnp.zeros((x.shape[0], 2 * HF - x.shape[1]), jnp.bfloat16)],
        axis=1)


def _unpad_bf16(xb):
    return xb[:, :H].astype(jnp.float32)


def _dis_of(p_ref):
    return lax.rsqrt(1.0 + p_ref[0, :] + p_ref[1, :])


def _k1_body(p_ref, x_ref, w1_ref, u_ref):
    dis = _dis_of(p_ref)
    hw = x_ref[:, 0:1] * w1_ref[0:1, :] + x_ref[:, 1:2] * w1_ref[1:2, :]
    u_ref[...] = _pad_bf16(dis[:, None] * hw)


def _k3_body(p_ref, m_ref, b1_ref, w2_ref, u_ref):
    dis = _dis_of(p_ref)
    h = jnp.maximum(dis[:, None] * _unpad_bf16(m_ref[...])
                    + b1_ref[0, :], 0.0)
    u_ref[...] = _pad_bf16(dis[:, None] * jnp.dot(
        h, w2_ref[...], preferred_element_type=jnp.float32))


def _k5_body(p_ref, m_ref, b2_ref, wl_ref, bl_ref, o_ref):
    dis = _dis_of(p_ref)
    h = jnp.maximum(dis[:, None] * _unpad_bf16(m_ref[...])
                    + b2_ref[0, :], 0.0)
    o = jnp.dot(h, wl_ref[...], preferred_element_type=jnp.float32)
    o = o + bl_ref[0, :]
    mx = jnp.maximum(o[:, 0:1], o[:, 1:2])
    lse = mx + jnp.log(jnp.exp(o[:, 0:1] - mx) + jnp.exp(o[:, 1:2] - mx))
    o_ref[...] = o - lse


def _row_spec(width):
    return pl.BlockSpec((_BLK, width), lambda i: (i, 0))


def _p_spec():
    return pl.BlockSpec((NC, _BLK), lambda i: (0, i))


def _full_spec(shape):
    return pl.BlockSpec(shape, lambda i: tuple(0 for _ in shape))


def _to_words(xb):
    return lax.bitcast_convert_type(xb.reshape(NP, HF, 2), jnp.float32)


def _to_bf(xw):
    return lax.bitcast_convert_type(xw, jnp.bfloat16).reshape(NP, 2 * HF)


def kernel(x, edge_index, W1, b1, W2, b2, Wlin, blin):
    row = edge_index[0]
    col = edge_index[1]
    x_pad = jnp.pad(x, ((0, NP - N), (0, 0)))

    p = _deg_kernel(col)

    u1 = pl.pallas_call(
        _k1_body,
        grid=(_GRID,),
        in_specs=[_p_spec(), _row_spec(2), _full_spec((2, H))],
        out_specs=_row_spec(2 * HF),
        out_shape=jax.ShapeDtypeStruct((NP, 2 * HF), jnp.bfloat16),
    )(p, x_pad, W1)
    u1 = _to_words(u1)

    m1 = _segmax_kernel(u1, col, row)

    u2 = pl.pallas_call(
        _k3_body,
        grid=(_GRID,),
        in_specs=[_p_spec(), _row_spec(2 * HF),
                  _full_spec((1, H)), _full_spec((H, H))],
        out_specs=_row_spec(2 * HF),
        out_shape=jax.ShapeDtypeStruct((NP, 2 * HF), jnp.bfloat16),
    )(p, _to_bf(m1), b1.reshape(1, H), W2)
    u2 = _to_words(u2)

    m2 = _segmax_kernel(u2, col, row)

    out = pl.pallas_call(
        _k5_body,
        grid=(_GRID,),
        in_specs=[_p_spec(), _row_spec(2 * HF),
                  _full_spec((1, H)), _full_spec((H, 2)), _full_spec((1, 2))],
        out_specs=_row_spec(2),
        out_shape=jax.ShapeDtypeStruct((NP, 2), jnp.float32),
    )(p, _to_bf(m2), b2.reshape(1, H), Wlin, blin.reshape(1, 2))

    return out[:N]
